# baseline (speedup 1.0000x reference)
import dataclasses
import functools

import jax
import jax.numpy as jnp
from jax import lax
from jax.experimental import pallas as pl
from jax.experimental.pallas import tpu as pltpu
from jax.experimental.pallas import tpu_sc as plsc

B, C, H, W = 16, 256, 32, 32
HW = H * W
K = 1024

_SC_INFO = plsc.get_sparse_core_info()
NC, NS, L = _SC_INFO.num_cores, _SC_INFO.num_subcores, _SC_INFO.num_lanes
NW = NC * NS
CPW = C // NW


def _match_kernel(x_ref, bank_ref, idx_ref, pn_ref):
    b = pl.program_id(0)

    @pl.when(b == 0)
    def _():
        bank = bank_ref[...]
        norm = jnp.sqrt(jnp.sum(bank * bank, axis=1, keepdims=True))
        pn_ref[...] = bank / jnp.maximum(norm, 1e-12)

    xb = x_ref[0].reshape(C, HW)
    xnorm = jnp.sqrt(jnp.sum(xb * xb, axis=0, keepdims=True))
    xn = xb / jnp.maximum(xnorm, 1e-12)

    sim = jnp.dot(pn_ref[...], xn, preferred_element_type=jnp.float32)

    m = jnp.max(sim, axis=0, keepdims=True)
    iota_k = lax.broadcasted_iota(jnp.int32, (K, HW), 0)
    masked = jnp.where(sim == m, iota_k, K)
    idx_ref[0] = jnp.min(masked, axis=0, keepdims=True)


def _match(x, bank):
    return pl.pallas_call(
        _match_kernel,
        grid=(B,),
        in_specs=[
            pl.BlockSpec((1, C, H, W), lambda b: (b, 0, 0, 0)),
            pl.BlockSpec((K, C), lambda b: (0, 0)),
        ],
        out_specs=pl.BlockSpec((1, 1, HW), lambda b: (b, 0, 0)),
        out_shape=jax.ShapeDtypeStruct((B, 1, HW), jnp.int32),
        scratch_shapes=[pltpu.VMEM((K, C), jnp.float32)],
    )(x, bank)


_SC_PARAMS = pltpu.CompilerParams()
if "needs_layout_passes" in pltpu.CompilerParams.__dataclass_fields__:
    _SC_PARAMS = dataclasses.replace(_SC_PARAMS, needs_layout_passes=False)


@functools.partial(
    pl.kernel,
    mesh=plsc.VectorSubcoreMesh(core_axis_name="c", subcore_axis_name="s"),
    compiler_params=_SC_PARAMS,
    out_type=jax.ShapeDtypeStruct((B, C, H, W), jnp.float32),
    scratch_types=[
        pltpu.VMEM((CPW, HW), jnp.float32),
        pltpu.VMEM((B, HW), jnp.int32),
        pltpu.VMEM((2, CPW, H, W), jnp.float32),
        pltpu.SemaphoreType.DMA,
    ],
)
def _sc_gather(bank_t_hbm, idx_hbm, out_hbm, brows, idxv, ostage, sem):
    wid = lax.axis_index("s") * NC + lax.axis_index("c")
    cbase = wid * CPW
    pltpu.sync_copy(bank_t_hbm.at[pl.ds(cbase, CPW)], brows)
    pltpu.sync_copy(idx_hbm, idxv)

    def _dma(b):
        par = b % 2
        return pltpu.make_async_copy(
            ostage.at[par], out_hbm.at[b, pl.ds(cbase, CPW)], sem)

    for b in range(B):
        par = b % 2
        if b >= 2:
            _dma(b - 2).wait()

        @pl.loop(0, H)
        def _(h):
            for wc in range(W // L):
                iv = idxv[b, pl.ds(h * W + wc * L, L)]
                for cl in range(CPW):
                    vals = plsc.load_gather(
                        brows, [jnp.full((L,), cl, jnp.int32), iv])
                    ostage[par, cl, h, pl.ds(wc * L, L)] = vals

        _dma(b).start()

    _dma(B - 2).wait()
    _dma(B - 1).wait()


def kernel(x, prototype_bank):
    idx3 = _match(x, prototype_bank)
    idx = idx3.reshape(B, HW)
    out = _sc_gather(prototype_bank.T, idx)
    return out, idx


# speedup vs baseline: 1.9970x; 1.9970x over previous
import dataclasses
import functools

import jax
import jax.numpy as jnp
from jax import lax
from jax.experimental import pallas as pl
from jax.experimental.pallas import tpu as pltpu
from jax.experimental.pallas import tpu_sc as plsc

B, C, H, W = 16, 256, 32, 32
HW = H * W
K = 1024

_SC_INFO = plsc.get_sparse_core_info()
NC, NS, L = _SC_INFO.num_cores, _SC_INFO.num_subcores, _SC_INFO.num_lanes
NW = NC * NS
CPW = C // NW


def _match_kernel(x_ref, bank_ref, idx_ref, pn_ref):
    b = pl.program_id(0)

    @pl.when(b == 0)
    def _():
        bank = bank_ref[...]
        norm = jnp.sqrt(jnp.sum(bank * bank, axis=1, keepdims=True))
        pn_ref[...] = bank / jnp.maximum(norm, 1e-12)

    xb = x_ref[0]
    xnorm = jnp.sqrt(jnp.sum(xb * xb, axis=0, keepdims=True))
    xn = xb / jnp.maximum(xnorm, 1e-12)

    sim = jnp.dot(pn_ref[...], xn, preferred_element_type=jnp.float32)

    m = jnp.max(sim, axis=0, keepdims=True)
    iota_k = lax.broadcasted_iota(jnp.int32, (K, HW), 0)
    masked = jnp.where(sim == m, iota_k, K)
    idx_ref[0] = jnp.min(masked, axis=0, keepdims=True)


def _match(xf, bank):
    return pl.pallas_call(
        _match_kernel,
        grid=(B,),
        in_specs=[
            pl.BlockSpec((1, C, HW), lambda b: (b, 0, 0)),
            pl.BlockSpec((K, C), lambda b: (0, 0)),
        ],
        out_specs=pl.BlockSpec((1, 1, HW), lambda b: (b, 0, 0)),
        out_shape=jax.ShapeDtypeStruct((B, 1, HW), jnp.int32),
        scratch_shapes=[pltpu.VMEM((K, C), jnp.float32)],
    )(xf, bank)


_SC_PARAMS = pltpu.CompilerParams()
if "needs_layout_passes" in pltpu.CompilerParams.__dataclass_fields__:
    _SC_PARAMS = dataclasses.replace(_SC_PARAMS, needs_layout_passes=False)

_JSTEP = 4


@functools.partial(
    pl.kernel,
    mesh=plsc.VectorSubcoreMesh(core_axis_name="c", subcore_axis_name="s"),
    compiler_params=_SC_PARAMS,
    out_type=jax.ShapeDtypeStruct((B, C, HW), jnp.float32),
    scratch_types=[
        pltpu.VMEM((CPW, HW), jnp.float32),
        pltpu.VMEM((B, HW), jnp.int32),
        pltpu.VMEM((2, CPW, HW), jnp.float32),
        pltpu.SemaphoreType.DMA,
    ],
)
def _sc_gather(bank_t_hbm, idx_hbm, out_hbm, brows, idxv, ostage, sem):
    wid = lax.axis_index("s") * NC + lax.axis_index("c")
    cbase = wid * CPW
    pltpu.sync_copy(bank_t_hbm.at[pl.ds(cbase, CPW)], brows)
    pltpu.sync_copy(idx_hbm, idxv)

    def _dma(b):
        return pltpu.make_async_copy(
            ostage.at[b % 2], out_hbm.at[b, pl.ds(cbase, CPW)], sem)

    for b in range(B):
        par = b % 2
        if b >= 2:
            _dma(b - 2).wait()

        @pl.loop(0, HW // L, step=_JSTEP)
        def _(j0):
            ivs = [idxv[b, pl.ds((j0 + u) * L, L)] for u in range(_JSTEP)]
            vals = [
                plsc.load_gather(brows, [jnp.full((L,), cl, jnp.int32), iv])
                for iv in ivs
                for cl in range(CPW)
            ]
            i = 0
            for u in range(_JSTEP):
                for cl in range(CPW):
                    ostage[par, cl, pl.ds((j0 + u) * L, L)] = vals[i]
                    i += 1

        _dma(b).start()

    _dma(B - 2).wait()
    _dma(B - 1).wait()


def kernel(x, prototype_bank):
    xf = x.reshape(B, C, HW)
    idx3 = _match(xf, prototype_bank)
    idx = idx3.reshape(B, HW)
    out = _sc_gather(prototype_bank.T, idx)
    return out.reshape(B, C, H, W), idx
